# baseline (speedup 1.0000x reference)
import jax
import jax.numpy as jnp
from jax.experimental import pallas as pl

N_NODES = 10000
HIDDEN = 128
N_PARENTS = 20000

BN = 1000
BM = 1024
BP = 2000
M_PAD = 126976


def _expand_body(x_ref, w_ref, t_ref):
    xb = x_ref[...]
    y = jnp.dot(xb, w_ref[...], preferred_element_type=jnp.float32)
    t_ref[0] = xb
    for j in range(7):
        t_ref[1 + j] = y[:, HIDDEN * j:HIDDEN * (j + 1)]
    t_ref[8] = jnp.zeros_like(xb)


def _expand(x, wcat):
    n = x.shape[0]
    return pl.pallas_call(
        _expand_body,
        grid=(n // BN,),
        in_specs=[
            pl.BlockSpec((BN, HIDDEN), lambda i: (i, 0)),
            pl.BlockSpec((HIDDEN, 7 * HIDDEN), lambda i: (0, 0)),
        ],
        out_specs=pl.BlockSpec((9, BN, HIDDEN), lambda i: (0, i, 0)),
        out_shape=jax.ShapeDtypeStruct((9, n, HIDDEN), jnp.float32),
    )(x, wcat)


def _child_body(b_ref, tm_ref, zf_ref, sf_ref, c_ref):
    b = b_ref[...]
    e = jnp.where(b > 0, b, jnp.exp(jnp.minimum(b, 0.0)) - 1.0)
    cz = jnp.dot(e, zf_ref[...], preferred_element_type=jnp.float32)
    cs = jnp.dot(e, sf_ref[...], preferred_element_type=jnp.float32)
    t = tm_ref[...]
    c_ref[...] = jnp.where(t == 2.0, cz, jnp.where(t == 3.0, cs, b))


def _child_finish(b, tm3, zft, sft):
    m = b.shape[0]
    return pl.pallas_call(
        _child_body,
        grid=(m // BM,),
        in_specs=[
            pl.BlockSpec((BM, HIDDEN), lambda i: (i, 0)),
            pl.BlockSpec((BM, 1), lambda i: (i, 0)),
            pl.BlockSpec((HIDDEN, HIDDEN), lambda i: (0, 0)),
            pl.BlockSpec((HIDDEN, HIDDEN), lambda i: (0, 0)),
        ],
        out_specs=pl.BlockSpec((BM, HIDDEN), lambda i: (i, 0)),
        out_shape=jax.ShapeDtypeStruct((m, HIDDEN), jnp.float32),
    )(b, tm3, zft, sft)


def _parent_body(s_ref, ta_ref, pf_ref, o_ref):
    s = s_ref[...]
    sp = jnp.dot(s, pf_ref[...], preferred_element_type=jnp.float32)
    e = jnp.where(s > 0, s, jnp.exp(jnp.minimum(s, 0.0)) - 1.0)
    t = ta_ref[...]
    o_ref[...] = jnp.where(t == 0.0, s, jnp.where(t == 1.0, sp, e))


def _parent_finish(s, ta3, pft):
    p = s.shape[0]
    return pl.pallas_call(
        _parent_body,
        grid=(p // BP,),
        in_specs=[
            pl.BlockSpec((BP, HIDDEN), lambda i: (i, 0)),
            pl.BlockSpec((BP, 1), lambda i: (i, 0)),
            pl.BlockSpec((HIDDEN, HIDDEN), lambda i: (0, 0)),
        ],
        out_specs=pl.BlockSpec((BP, HIDDEN), lambda i: (i, 0)),
        out_shape=jax.ShapeDtypeStruct((p, HIDDEN), jnp.float32),
    )(s, ta3, pft)


def kernel(x, p_w, p_final_w, z_w, z_final_w, s_w, s_final_w,
           initial_map, order_matrix, pooling, type_mask):
    n, h = x.shape
    k, m = order_matrix.shape
    p = N_PARENTS
    im = initial_map.astype(jnp.int32)
    om = order_matrix.astype(jnp.int32)
    tm = type_mask.astype(jnp.int32)
    pool = pooling.astype(jnp.int32)

    zrow = 8 * n
    base0 = jnp.where(tm == 0, 0,
            jnp.where(tm == 1, n,
            jnp.where(tm == 2, 2 * n, 5 * n)))
    g0 = base0 + im

    def gk(kk):
        omk = om[kk]
        valid = (omk >= 0) & (tm >= 2)
        imk = im[jnp.clip(omk, 0, m - 1)]
        base = jnp.where(tm == 2, (2 + kk) * n, (5 + kk) * n)
        return jnp.where(valid, base + imk, zrow)

    pad = M_PAD - m
    zpad = jnp.full((pad,), zrow, jnp.int32)
    g0 = jnp.concatenate([g0, zpad])
    g1 = jnp.concatenate([gk(1), zpad])
    g2 = jnp.concatenate([gk(2), zpad])
    tm_pad = jnp.concatenate([tm, jnp.zeros((pad,), jnp.int32)])
    tm3 = tm_pad.astype(jnp.float32).reshape(M_PAD, 1)

    starts = jnp.searchsorted(pool, jnp.arange(p)).astype(jnp.int32)
    tm_after = tm[starts]
    cnt = jnp.where(tm_after == 0, 1, 8)
    jj = jnp.arange(8, dtype=jnp.int32)[:, None]
    gd = jnp.where(jj < cnt[None, :], starts[None, :] + jj, m)
    ta3 = tm_after.astype(jnp.float32).reshape(p, 1)

    wcat = jnp.concatenate([p_w.T] + [z_w[i].T for i in range(3)]
                           + [s_w[i].T for i in range(3)], axis=1)
    t_tab = _expand(x, wcat).reshape(9 * n, h)

    b = t_tab[g0] + t_tab[g1] + t_tab[g2]

    c = _child_finish(b, tm3, z_final_w.T, s_final_w.T)

    s = c[gd].sum(axis=0)

    return _parent_finish(s, ta3, p_final_w.T)
